# baseline (speedup 1.0000x reference)
import functools

import jax
import jax.numpy as jnp
from jax import lax
from jax.experimental import pallas as pl
from jax.experimental.pallas import tpu as pltpu
from jax.experimental.pallas import tpu_sc as plsc

M, A, AEV, NS = 128, 64, 384, 4
N = M * A
P = 131072
H0, NB = 96, 48
CUTOFF = 5.2

BLK = 4096
GRID = N // BLK
DBLK = P // GRID

NCORES, NSUB = 2, 16
NW = NCORES * NSUB
NMSG = 2 * P
MSG_PER_W = NMSG // NW
CHUNK = 128
NCHUNK = MSG_PER_W // CHUNK
ROWS_PER_TILE = N // NSUB


def _species_sel(x, sp, W_ref):
    acc = None
    for s in range(NS):
        h = jnp.dot(x, W_ref[s], preferred_element_type=jnp.float32)
        term = jnp.where(sp == s, h, 0.0)
        acc = term if acc is None else acc + term
    return acc


def _species_mlp(x, sp, W_ref):
    return jax.nn.gelu(_species_sel(x, sp, W_ref))


def _tc0_body(pf_ref, fac_ref, sp_ref, aev_ref, dist_ref, Wp_ref, Wn_ref,
              int_ref, nb_ref, dec_ref):
    x = aev_ref[...]
    sp = sp_ref[...]
    internal = _species_mlp(x, sp, Wp_ref)
    int_ref[...] = internal
    nb_ref[...] = _species_mlp(internal, sp, Wn_ref)
    d = dist_ref[...]
    u = jnp.clip(1.0 - d * (1.0 / CUTOFF), 0.0, 1.0)
    smooth = u * u * (3.0 - 2.0 * u)
    pf = pf_ref[0, 0]
    fac = fac_ref[0, 0]
    dec_ref[...] = (pf * pf) * jnp.exp(-(fac * fac) * d) * smooth


def _tc_round0(pf, fac, sp, aev, dist3, Wp0, Wn0):
    return pl.pallas_call(
        _tc0_body,
        grid=(GRID,),
        in_specs=[
            pl.BlockSpec(memory_space=pltpu.SMEM),
            pl.BlockSpec(memory_space=pltpu.SMEM),
            pl.BlockSpec((BLK, 1), lambda i: (i, 0)),
            pl.BlockSpec((BLK, AEV), lambda i: (i, 0)),
            pl.BlockSpec((1, 1, DBLK), lambda i: (i, 0, 0)),
            pl.BlockSpec((NS, AEV, H0), lambda i: (0, 0, 0)),
            pl.BlockSpec((NS, H0, NB), lambda i: (0, 0, 0)),
        ],
        out_specs=[
            pl.BlockSpec((BLK, H0), lambda i: (i, 0)),
            pl.BlockSpec((BLK, NB), lambda i: (i, 0)),
            pl.BlockSpec((1, 1, DBLK), lambda i: (i, 0, 0)),
        ],
        out_shape=[
            jax.ShapeDtypeStruct((N, H0), jnp.float32),
            jax.ShapeDtypeStruct((N, NB), jnp.float32),
            jax.ShapeDtypeStruct((GRID, 1, DBLK), jnp.float32),
        ],
    )(pf, fac, sp, aev, dist3, Wp0, Wn0)


def _tc1_body(sp_ref, int_ref, mg_ref, Wp_ref, Wn_ref, int_out, nb_out):
    sp = sp_ref[...]
    x = jnp.concatenate([int_ref[...], mg_ref[0] + mg_ref[1]], axis=-1)
    internal = _species_mlp(x, sp, Wp_ref)
    int_out[...] = internal
    nb_out[...] = _species_mlp(internal, sp, Wn_ref)


def _tc_round1(sp, internal, merged, Wp1, Wn1):
    return pl.pallas_call(
        _tc1_body,
        grid=(GRID,),
        in_specs=[
            pl.BlockSpec((BLK, 1), lambda i: (i, 0)),
            pl.BlockSpec((BLK, H0), lambda i: (i, 0)),
            pl.BlockSpec((2, BLK, NB), lambda i: (0, i, 0)),
            pl.BlockSpec((NS, H0 + NB, H0), lambda i: (0, 0, 0)),
            pl.BlockSpec((NS, H0, NB), lambda i: (0, 0, 0)),
        ],
        out_specs=[
            pl.BlockSpec((BLK, H0), lambda i: (i, 0)),
            pl.BlockSpec((BLK, NB), lambda i: (i, 0)),
        ],
        out_shape=[
            jax.ShapeDtypeStruct((N, H0), jnp.float32),
            jax.ShapeDtypeStruct((N, NB), jnp.float32),
        ],
    )(sp, internal, merged, Wp1, Wn1)


def _tcf_body(sp_ref, int_ref, mg_ref, tch_ref, Wf1_ref, Wf2_ref, out_ref):
    sp = sp_ref[...]
    x = jnp.concatenate([int_ref[...], mg_ref[0] + mg_ref[1]], axis=-1)
    hid = _species_mlp(x, sp, Wf1_ref)
    pre = None
    for s in range(NS):
        h = jnp.dot(hid, Wf2_ref[s], preferred_element_type=jnp.float32)
        term = jnp.where(sp == s, h, 0.0)
        pre = term if pre is None else pre + term
    nmol = BLK // A
    row_mol = lax.broadcasted_iota(jnp.int32, (BLK, nmol), 0) // A
    col_mol = lax.broadcasted_iota(jnp.int32, (BLK, nmol), 1)
    mask = row_mol == col_mol
    sums = jnp.sum(jnp.where(mask, pre, 0.0), axis=0)
    excess = tch_ref[0, 0] - sums
    add = jnp.sum(jnp.where(mask, excess[None, :], 0.0), axis=1,
                  keepdims=True)
    out_ref[...] = pre + add * (1.0 / A)


def _tc_final(sp, internal, merged, tch3, Wf1, Wf2):
    nmol = BLK // A
    return pl.pallas_call(
        _tcf_body,
        grid=(GRID,),
        in_specs=[
            pl.BlockSpec((BLK, 1), lambda i: (i, 0)),
            pl.BlockSpec((BLK, H0), lambda i: (i, 0)),
            pl.BlockSpec((2, BLK, NB), lambda i: (0, i, 0)),
            pl.BlockSpec((1, 1, nmol), lambda i: (i, 0, 0)),
            pl.BlockSpec((NS, H0 + NB, H0), lambda i: (0, 0, 0)),
            pl.BlockSpec((NS, H0, 1), lambda i: (0, 0, 0)),
        ],
        out_specs=pl.BlockSpec((BLK, 1), lambda i: (i, 0)),
        out_shape=jax.ShapeDtypeStruct((N, 1), jnp.float32),
    )(sp, internal, merged, tch3, Wf1, Wf2)


NBUF = 8
AHEAD = 4
KPW = NCHUNK


def _sc_body(ai3, dec3, nbtab, out, dsti, srci, decv, rows, table,
             gsem, ssem):
    cid = lax.axis_index("c")
    sid = lax.axis_index("s")
    wid = sid * NCORES + cid
    r = wid // NSUB
    q = wid % NSUB

    pltpu.sync_copy(ai3.at[r, pl.ds(q * KPW, KPW)], dsti)
    pltpu.sync_copy(ai3.at[1 - r, pl.ds(q * KPW, KPW)], srci)
    pltpu.sync_copy(dec3.at[pl.ds(q * KPW, KPW)], decv)

    zv = jnp.zeros((16,), jnp.float32)

    def zero_rows(e, carry):
        for j in range(NB // 16):
            rows[0, e, pl.ds(j * 16, 16)] = zv
        return carry

    lax.fori_loop(0, CHUNK, zero_rows, 0)
    for t in range(ROWS_PER_TILE // CHUNK):
        pltpu.sync_copy(rows.at[0], table.at[pl.ds(
            sid * ROWS_PER_TILE + t * CHUNK, CHUNK)])
    plsc.subcore_barrier()

    def gather_start(k, b):
        pltpu.make_async_copy(nbtab.at[srci.at[k]], rows.at[b],
                              gsem.at[b]).start()

    def gather_wait(k, b):
        pltpu.make_async_copy(nbtab.at[srci.at[k]], rows.at[b],
                              gsem.at[b]).wait()

    def scatter_start(k, b):
        pltpu.async_copy(rows.at[b], table.at[dsti.at[k]], ssem.at[b],
                         add=True)

    def scatter_wait(k, b):
        pltpu.make_async_copy(rows.at[b], table.at[dsti.at[k]],
                              ssem.at[b]).wait()

    for b in range(AHEAD):
        gather_start(b, b)

    def super_body(K, carry):
        for b in range(NBUF):
            k = K * NBUF + b
            bn = (b + AHEAD) % NBUF
            kn = k + AHEAD

            @pl.when(kn >= NBUF)
            def _():
                scatter_wait(kn - NBUF, bn)

            @pl.when(kn < KPW)
            def _():
                gather_start(kn, bn)

            gather_wait(k, b)

            def scale(g, c2):
                dv = decv[k, pl.ds(g * 16, 16)]
                for j in range(16):
                    d = dv[j]
                    e = g * 16 + j
                    for k2 in range(NB // 16):
                        rows[b, e, pl.ds(k2 * 16, 16)] = (
                            rows[b, e, pl.ds(k2 * 16, 16)] * d)
                return c2

            lax.fori_loop(0, CHUNK // 16, scale, 0)
            scatter_start(k, b)
        return carry

    lax.fori_loop(0, KPW // NBUF, super_body, 0)
    for kk in range(KPW - (NBUF - AHEAD), KPW):
        scatter_wait(kk, kk % NBUF)
    plsc.subcore_barrier()

    pltpu.sync_copy(table.at[pl.ds(sid * ROWS_PER_TILE, ROWS_PER_TILE)],
                    out.at[cid, pl.ds(sid * ROWS_PER_TILE, ROWS_PER_TILE)])


@functools.cache
def _get_sc_scatter():
    return functools.partial(
        pl.kernel,
        mesh=plsc.VectorSubcoreMesh(core_axis_name="c", subcore_axis_name="s"),
        compiler_params=pltpu.CompilerParams(use_tc_tiling_on_sc=False),
        out_type=jax.ShapeDtypeStruct((NCORES, N, NB), jnp.float32),
        scratch_types=[
            pltpu.VMEM((KPW, CHUNK), jnp.int32),
            pltpu.VMEM((KPW, CHUNK), jnp.int32),
            pltpu.VMEM((KPW, CHUNK), jnp.float32),
            pltpu.VMEM((NBUF, CHUNK, NB), jnp.float32),
            pltpu.VMEM_SHARED((N, NB), jnp.float32),
            pltpu.SemaphoreType.DMA((NBUF,)),
            pltpu.SemaphoreType.DMA((NBUF,)),
        ],
    )(_sc_body)


def _sc_scatter(ai12, decay, nbtab):
    ai3 = ai12.reshape(2, P // CHUNK, CHUNK)
    dec3 = decay.reshape(P // CHUNK, CHUNK)
    return _get_sc_scatter()(ai3, dec3, nbtab)


def kernel(species, aev, atom_index12, distances, total_charges,
           Wp0, Wn0, Wp1, Wn1, Wf1, Wf2, prefactor, factor):
    sp = species.reshape(N, 1).astype(jnp.int32)
    ai12 = atom_index12.astype(jnp.int32)
    dist3 = distances.astype(jnp.float32).reshape(GRID, 1, DBLK)
    tch3 = total_charges.astype(jnp.float32).reshape(GRID, 1, BLK // A)
    pf = jnp.asarray(prefactor, jnp.float32).reshape(1, 1)
    fac = jnp.asarray(factor, jnp.float32).reshape(1, 1)
    feats = aev.reshape(N, AEV).astype(jnp.float32)

    internal0, nb0, dec3 = _tc_round0(pf, fac, sp, feats, dist3, Wp0, Wn0)
    decay = dec3.reshape(P)
    merged0 = _sc_scatter(ai12, decay, nb0)
    internal1, nb1 = _tc_round1(sp, internal0, merged0, Wp1, Wn1)
    merged1 = _sc_scatter(ai12, decay, nb1)
    pre = _tc_final(sp, internal1, merged1, tch3, Wf1, Wf2)
    return pre.reshape(M, A)


# speedup vs baseline: 1.0541x; 1.0541x over previous
import functools

import jax
import jax.numpy as jnp
from jax import lax
from jax.experimental import pallas as pl
from jax.experimental.pallas import tpu as pltpu
from jax.experimental.pallas import tpu_sc as plsc

M, A, AEV, NS = 128, 64, 384, 4
N = M * A
P = 131072
H0, NB = 96, 48
CUTOFF = 5.2

BLK = 2048
GRID = N // BLK
DBLK = P // GRID

NCORES, NSUB = 2, 16
NW = NCORES * NSUB
NMSG = 2 * P
MSG_PER_W = NMSG // NW
CHUNK = 128
NCHUNK = MSG_PER_W // CHUNK
ROWS_PER_TILE = N // NSUB


def _species_sel(x, sp, W_ref):
    acc = None
    for s in range(NS):
        h = jnp.dot(x, W_ref[s], preferred_element_type=jnp.float32)
        term = jnp.where(sp == s, h, 0.0)
        acc = term if acc is None else acc + term
    return acc


def _species_mlp(x, sp, W_ref):
    return jax.nn.gelu(_species_sel(x, sp, W_ref))


def _tc0_body(pf_ref, fac_ref, sp_ref, aev_ref, dist_ref, Wp_ref, Wn_ref,
              int_ref, nb_ref, dec_ref):
    x = aev_ref[...]
    sp = sp_ref[...]
    internal = _species_mlp(x, sp, Wp_ref)
    int_ref[...] = internal
    nb_ref[...] = _species_mlp(internal, sp, Wn_ref)
    d = dist_ref[...]
    u = jnp.clip(1.0 - d * (1.0 / CUTOFF), 0.0, 1.0)
    smooth = u * u * (3.0 - 2.0 * u)
    pf = pf_ref[0, 0]
    fac = fac_ref[0, 0]
    dec_ref[...] = (pf * pf) * jnp.exp(-(fac * fac) * d) * smooth


def _tc_round0(pf, fac, sp, aev, dist3, Wp0, Wn0):
    return pl.pallas_call(
        _tc0_body,
        grid=(GRID,),
        in_specs=[
            pl.BlockSpec(memory_space=pltpu.SMEM),
            pl.BlockSpec(memory_space=pltpu.SMEM),
            pl.BlockSpec((BLK, 1), lambda i: (i, 0)),
            pl.BlockSpec((BLK, AEV), lambda i: (i, 0)),
            pl.BlockSpec((1, 1, DBLK), lambda i: (i, 0, 0)),
            pl.BlockSpec((NS, AEV, H0), lambda i: (0, 0, 0)),
            pl.BlockSpec((NS, H0, NB), lambda i: (0, 0, 0)),
        ],
        out_specs=[
            pl.BlockSpec((BLK, H0), lambda i: (i, 0)),
            pl.BlockSpec((BLK, NB), lambda i: (i, 0)),
            pl.BlockSpec((1, 1, DBLK), lambda i: (i, 0, 0)),
        ],
        out_shape=[
            jax.ShapeDtypeStruct((N, H0), jnp.float32),
            jax.ShapeDtypeStruct((N, NB), jnp.float32),
            jax.ShapeDtypeStruct((GRID, 1, DBLK), jnp.float32),
        ],
    )(pf, fac, sp, aev, dist3, Wp0, Wn0)


def _tc1_body(sp_ref, int_ref, mg_ref, Wp_ref, Wn_ref, int_out, nb_out):
    sp = sp_ref[...]
    x = jnp.concatenate([int_ref[...], mg_ref[0] + mg_ref[1]], axis=-1)
    internal = _species_mlp(x, sp, Wp_ref)
    int_out[...] = internal
    nb_out[...] = _species_mlp(internal, sp, Wn_ref)


def _tc_round1(sp, internal, merged, Wp1, Wn1):
    return pl.pallas_call(
        _tc1_body,
        grid=(GRID,),
        in_specs=[
            pl.BlockSpec((BLK, 1), lambda i: (i, 0)),
            pl.BlockSpec((BLK, H0), lambda i: (i, 0)),
            pl.BlockSpec((2, BLK, NB), lambda i: (0, i, 0)),
            pl.BlockSpec((NS, H0 + NB, H0), lambda i: (0, 0, 0)),
            pl.BlockSpec((NS, H0, NB), lambda i: (0, 0, 0)),
        ],
        out_specs=[
            pl.BlockSpec((BLK, H0), lambda i: (i, 0)),
            pl.BlockSpec((BLK, NB), lambda i: (i, 0)),
        ],
        out_shape=[
            jax.ShapeDtypeStruct((N, H0), jnp.float32),
            jax.ShapeDtypeStruct((N, NB), jnp.float32),
        ],
    )(sp, internal, merged, Wp1, Wn1)


def _tcf_body(sp_ref, int_ref, mg_ref, tch_ref, Wf1_ref, Wf2_ref, out_ref):
    sp = sp_ref[...]
    x = jnp.concatenate([int_ref[...], mg_ref[0] + mg_ref[1]], axis=-1)
    hid = _species_mlp(x, sp, Wf1_ref)
    pre = None
    for s in range(NS):
        h = jnp.dot(hid, Wf2_ref[s], preferred_element_type=jnp.float32)
        term = jnp.where(sp == s, h, 0.0)
        pre = term if pre is None else pre + term
    nmol = BLK // A
    row_mol = lax.broadcasted_iota(jnp.int32, (BLK, nmol), 0) // A
    col_mol = lax.broadcasted_iota(jnp.int32, (BLK, nmol), 1)
    mask = row_mol == col_mol
    sums = jnp.sum(jnp.where(mask, pre, 0.0), axis=0)
    excess = tch_ref[0, 0] - sums
    add = jnp.sum(jnp.where(mask, excess[None, :], 0.0), axis=1,
                  keepdims=True)
    out_ref[...] = pre + add * (1.0 / A)


def _tc_final(sp, internal, merged, tch3, Wf1, Wf2):
    nmol = BLK // A
    return pl.pallas_call(
        _tcf_body,
        grid=(GRID,),
        in_specs=[
            pl.BlockSpec((BLK, 1), lambda i: (i, 0)),
            pl.BlockSpec((BLK, H0), lambda i: (i, 0)),
            pl.BlockSpec((2, BLK, NB), lambda i: (0, i, 0)),
            pl.BlockSpec((1, 1, nmol), lambda i: (i, 0, 0)),
            pl.BlockSpec((NS, H0 + NB, H0), lambda i: (0, 0, 0)),
            pl.BlockSpec((NS, H0, 1), lambda i: (0, 0, 0)),
        ],
        out_specs=pl.BlockSpec((BLK, 1), lambda i: (i, 0)),
        out_shape=jax.ShapeDtypeStruct((N, 1), jnp.float32),
    )(sp, internal, merged, tch3, Wf1, Wf2)


NBUF = 8
AHEAD = 4
KPW = NCHUNK


def _sc_body(ai3, dec3, nbtab, out, dsti, srci, decv, rows, table,
             gsem, ssem):
    cid = lax.axis_index("c")
    sid = lax.axis_index("s")
    wid = sid * NCORES + cid
    r = wid // NSUB
    q = wid % NSUB

    pre_d = pltpu.make_async_copy(ai3.at[r, pl.ds(q * KPW, KPW)], dsti,
                                  gsem.at[0])
    pre_s = pltpu.make_async_copy(ai3.at[1 - r, pl.ds(q * KPW, KPW)], srci,
                                  gsem.at[1])
    pre_v = pltpu.make_async_copy(dec3.at[pl.ds(q * KPW, KPW)], decv,
                                  gsem.at[2])
    pre_d.start()
    pre_s.start()
    pre_v.start()

    zv = jnp.zeros((16,), jnp.float32)

    def zero_rows(e, carry):
        for j in range(NB // 16):
            rows[0, e, pl.ds(j * 16, 16)] = zv
        return carry

    lax.fori_loop(0, CHUNK, zero_rows, 0)
    for t in range(ROWS_PER_TILE // CHUNK):
        pltpu.async_copy(rows.at[0], table.at[pl.ds(
            sid * ROWS_PER_TILE + t * CHUNK, CHUNK)], ssem.at[t])
    pre_d.wait()
    pre_s.wait()
    pre_v.wait()
    for t in range(ROWS_PER_TILE // CHUNK):
        pltpu.make_async_copy(rows.at[0], table.at[pl.ds(
            sid * ROWS_PER_TILE + t * CHUNK, CHUNK)], ssem.at[t]).wait()
    plsc.subcore_barrier()

    def gather_start(k, b):
        pltpu.make_async_copy(nbtab.at[srci.at[k]], rows.at[b],
                              gsem.at[b]).start()

    def gather_wait(k, b):
        pltpu.make_async_copy(nbtab.at[srci.at[k]], rows.at[b],
                              gsem.at[b]).wait()

    def scatter_start(k, b):
        pltpu.async_copy(rows.at[b], table.at[dsti.at[k]], ssem.at[b],
                         add=True)

    def scatter_wait(k, b):
        pltpu.make_async_copy(rows.at[b], table.at[dsti.at[k]],
                              ssem.at[b]).wait()

    for b in range(AHEAD):
        gather_start(b, b)

    def super_body(K, carry):
        for b in range(NBUF):
            k = K * NBUF + b
            bn = (b + AHEAD) % NBUF
            kn = k + AHEAD

            @pl.when(kn >= NBUF)
            def _():
                scatter_wait(kn - NBUF, bn)

            @pl.when(kn < KPW)
            def _():
                gather_start(kn, bn)

            gather_wait(k, b)

            def scale(g, c2):
                dv = decv[k, pl.ds(g * 16, 16)]
                for j in range(16):
                    d = dv[j]
                    e = g * 16 + j
                    for k2 in range(NB // 16):
                        rows[b, e, pl.ds(k2 * 16, 16)] = (
                            rows[b, e, pl.ds(k2 * 16, 16)] * d)
                return c2

            lax.fori_loop(0, CHUNK // 16, scale, 0)
            scatter_start(k, b)
        return carry

    lax.fori_loop(0, KPW // NBUF, super_body, 0)
    for kk in range(KPW - (NBUF - AHEAD), KPW):
        scatter_wait(kk, kk % NBUF)
    plsc.subcore_barrier()

    pltpu.sync_copy(table.at[pl.ds(sid * ROWS_PER_TILE, ROWS_PER_TILE)],
                    out.at[cid, pl.ds(sid * ROWS_PER_TILE, ROWS_PER_TILE)])


@functools.cache
def _get_sc_scatter():
    return functools.partial(
        pl.kernel,
        mesh=plsc.VectorSubcoreMesh(core_axis_name="c", subcore_axis_name="s"),
        compiler_params=pltpu.CompilerParams(use_tc_tiling_on_sc=False),
        out_type=jax.ShapeDtypeStruct((NCORES, N, NB), jnp.float32),
        scratch_types=[
            pltpu.VMEM((KPW, CHUNK), jnp.int32),
            pltpu.VMEM((KPW, CHUNK), jnp.int32),
            pltpu.VMEM((KPW, CHUNK), jnp.float32),
            pltpu.VMEM((NBUF, CHUNK, NB), jnp.float32),
            pltpu.VMEM_SHARED((N, NB), jnp.float32),
            pltpu.SemaphoreType.DMA((NBUF,)),
            pltpu.SemaphoreType.DMA((NBUF,)),
        ],
    )(_sc_body)


def _sc_scatter(ai12, decay, nbtab):
    ai3 = ai12.reshape(2, P // CHUNK, CHUNK)
    dec3 = decay.reshape(P // CHUNK, CHUNK)
    return _get_sc_scatter()(ai3, dec3, nbtab)


def kernel(species, aev, atom_index12, distances, total_charges,
           Wp0, Wn0, Wp1, Wn1, Wf1, Wf2, prefactor, factor):
    sp = species.reshape(N, 1).astype(jnp.int32)
    ai12 = atom_index12.astype(jnp.int32)
    dist3 = distances.astype(jnp.float32).reshape(GRID, 1, DBLK)
    tch3 = total_charges.astype(jnp.float32).reshape(GRID, 1, BLK // A)
    pf = jnp.asarray(prefactor, jnp.float32).reshape(1, 1)
    fac = jnp.asarray(factor, jnp.float32).reshape(1, 1)
    feats = aev.reshape(N, AEV).astype(jnp.float32)

    internal0, nb0, dec3 = _tc_round0(pf, fac, sp, feats, dist3, Wp0, Wn0)
    decay = dec3.reshape(P)
    merged0 = _sc_scatter(ai12, decay, nb0)
    internal1, nb1 = _tc_round1(sp, internal0, merged0, Wp1, Wn1)
    merged1 = _sc_scatter(ai12, decay, nb1)
    pre = _tc_final(sp, internal1, merged1, tch3, Wf1, Wf2)
    return pre.reshape(M, A)
